# baseline (speedup 1.0000x reference)
import functools
import jax
import jax.numpy as jnp
from jax import lax
from jax.experimental import pallas as pl
from jax.experimental.pallas import tpu as pltpu, tpu_sc as plsc

_F32 = jnp.float32


def _node_precompute(h, Wm_h, wa_sd):
    n, d = h.shape
    nb = 2000

    def body(h_ref, w_ref, wsd_ref, hm_ref, asd_ref):
        hb = h_ref[...]
        hm_ref[...] = jnp.dot(hb, w_ref[...], preferred_element_type=_F32)
        asd_ref[...] = jnp.dot(hb, wsd_ref[...], preferred_element_type=_F32)

    return pl.pallas_call(
        body,
        grid=(n // nb,),
        in_specs=[
            pl.BlockSpec((nb, d), lambda i: (i, 0)),
            pl.BlockSpec((d, d), lambda i: (0, 0)),
            pl.BlockSpec((d, 2), lambda i: (0, 0)),
        ],
        out_specs=[
            pl.BlockSpec((nb, d), lambda i: (i, 0)),
            pl.BlockSpec((nb, 2), lambda i: (i, 0)),
        ],
        out_shape=[
            jax.ShapeDtypeStruct((n, d), _F32),
            jax.ShapeDtypeStruct((n, 2), _F32),
        ],
    )(h, Wm_h, wa_sd)


def _edge_precompute(eft, We_m, wa_e, bm, ba, d):
    de, e = eft.shape
    eb = 16000

    def body(ef_ref, w_ref, we_ref, bm_ref, ba_ref, em_ref, ae_ref):
        efb = ef_ref[...]
        em_ref[...] = lax.dot_general(
            efb, w_ref[...], (((0,), (0,)), ((), ())),
            preferred_element_type=_F32) + bm_ref[...]
        ae_ref[...] = lax.dot_general(
            we_ref[...], efb, (((0,), (0,)), ((), ())),
            preferred_element_type=_F32) + ba_ref[...]

    return pl.pallas_call(
        body,
        grid=(e // eb,),
        in_specs=[
            pl.BlockSpec((de, eb), lambda i: (0, i)),
            pl.BlockSpec((de, d), lambda i: (0, 0)),
            pl.BlockSpec((de, 1), lambda i: (0, 0)),
            pl.BlockSpec((1, d), lambda i: (0, 0)),
            pl.BlockSpec((1, 1), lambda i: (0, 0)),
        ],
        out_specs=[
            pl.BlockSpec((eb, d), lambda i: (i, 0)),
            pl.BlockSpec((1, eb), lambda i: (0, i)),
        ],
        out_shape=[
            jax.ShapeDtypeStruct((e, d), _F32),
            jax.ShapeDtypeStruct((1, e), _F32),
        ],
    )(eft, We_m, wa_e, bm, ba)


def _ln_relu(parts, gamma, beta):
    _, n, d = parts.shape
    nb = 2000

    def body(p_ref, g_ref, b_ref, o_ref):
        x = p_ref[0] + p_ref[1]
        m = jnp.mean(x, axis=-1, keepdims=True)
        c = x - m
        v = jnp.mean(c * c, axis=-1, keepdims=True)
        y = c * lax.rsqrt(v + 1e-5) * g_ref[...] + b_ref[...]
        o_ref[...] = jnp.maximum(y, 0.0)

    return pl.pallas_call(
        body,
        grid=(n // nb,),
        in_specs=[
            pl.BlockSpec((2, nb, d), lambda i: (0, i, 0)),
            pl.BlockSpec((1, d), lambda i: (0, 0)),
            pl.BlockSpec((1, d), lambda i: (0, 0)),
        ],
        out_specs=pl.BlockSpec((nb, d), lambda i: (i, 0)),
        out_shape=jax.ShapeDtypeStruct((n, d), _F32),
    )(parts, gamma.reshape(1, d), beta.reshape(1, d))


def _sc_edge_agg(hm, em, src, dst, ae, a_s, a_d):
    n, d = hm.shape
    ne = em.shape[0]
    NC, NS, L = 2, 16, 16
    NW = NC * NS
    ept = ne // NW
    CB = 80
    nch = ept // CB
    ZR = 32
    npad = ((n + NS * ZR - 1) // (NS * ZR)) * NS * ZR
    rpt = npad // NS

    mesh = plsc.VectorSubcoreMesh(core_axis_name="c", subcore_axis_name="s")

    buf_set = [
        pltpu.VMEM((CB,), jnp.int32),
        pltpu.VMEM((CB,), jnp.int32),
        pltpu.VMEM((CB,), _F32),
        pltpu.VMEM((CB,), _F32),
        pltpu.VMEM((CB,), _F32),
        pltpu.VMEM((CB,), _F32),
        pltpu.VMEM((CB, d), _F32),
        pltpu.VMEM((CB, d), _F32),
        pltpu.VMEM((CB,), jnp.int32),
        pltpu.SemaphoreType.DMA,
        pltpu.SemaphoreType.DMA,
        pltpu.SemaphoreType.DMA,
    ]

    @functools.partial(
        pl.kernel,
        out_type=jax.ShapeDtypeStruct((NC, npad, d), _F32),
        mesh=mesh,
        compiler_params=pltpu.CompilerParams(needs_layout_passes=False),
        scratch_types=buf_set + buf_set + [
            pltpu.VMEM_SHARED((npad, d), _F32),
        ],
    )
    def k(hm_hbm, em_hbm, src_hbm, dst_hbm, ae_hbm, as_hbm, ad_hbm, out_hbm,
          *refs):
        A, B, agg = refs[:12], refs[12:24], refs[24]
        cid = lax.axis_index("c")
        sid = lax.axis_index("s")
        wid = sid * NC + cid
        ebase = wid * ept

        def issue_meta(X, c):
            srcX, dstX, aeX, _, _, _, _, emX, _, msX, _, _ = X
            off = ebase + c * CB
            pltpu.async_copy(src_hbm.at[pl.ds(off, CB)], srcX, msX)
            pltpu.async_copy(dst_hbm.at[pl.ds(off, CB)], dstX, msX)
            pltpu.async_copy(ae_hbm.at[pl.ds(off, CB)], aeX, msX)
            pltpu.async_copy(em_hbm.at[pl.ds(off, CB)], emX, msX)

        def wait_meta(X, c):
            srcX, dstX, aeX, _, _, _, _, emX, _, msX, _, _ = X
            off = ebase + c * CB
            pltpu.make_async_copy(src_hbm.at[pl.ds(off, CB)], srcX, msX).wait()
            pltpu.make_async_copy(dst_hbm.at[pl.ds(off, CB)], dstX, msX).wait()
            pltpu.make_async_copy(ae_hbm.at[pl.ds(off, CB)], aeX, msX).wait()
            pltpu.make_async_copy(em_hbm.at[pl.ds(off, CB)], emX, msX).wait()

        def issue_gather(X):
            srcX, dstX, _, asX, adX, _, gX, _, _, _, gsX, _ = X
            pltpu.async_copy(hm_hbm.at[srcX], gX, gsX)
            pltpu.async_copy(as_hbm.at[srcX], asX, gsX)
            pltpu.async_copy(ad_hbm.at[dstX], adX, gsX)

        def wait_gather(X):
            srcX, dstX, _, asX, adX, _, gX, _, _, _, gsX, _ = X
            pltpu.make_async_copy(hm_hbm.at[srcX], gX, gsX).wait()
            pltpu.make_async_copy(as_hbm.at[srcX], asX, gsX).wait()
            pltpu.make_async_copy(ad_hbm.at[dstX], adX, gsX).wait()

        def wait_scatter(X):
            _, _, _, _, _, _, gX, _, dssX, _, _, ssX = X
            pltpu.make_async_copy(gX, agg.at[dssX], ssX).wait()

        def half(X, Y, c, do_meta, do_next, wait_prev_scatter=True):
            srcX, dstX, aeX, asX, adX, alX, gX, emX, dssX, msX, gsX, ssX = X
            with jax.named_scope("wg"):
                wait_gather(X)

            with jax.named_scope("alpha"):
                def ablk(j, _):
                    sl = pl.ds(j * L, L)
                    logit = asX[sl] + adX[sl] + aeX[sl]
                    alX[sl] = 1.0 / (1.0 + jnp.exp(-logit))
                    return 0
                lax.fori_loop(0, CB // L, ablk, 0)

            with jax.named_scope("erow"):
                def erow(r, _):
                    es = [r * 2 + u for u in range(2)]
                    abs_ = [plsc.load_gather(alX,
                                             [jnp.full((L,), e, jnp.int32)])
                            for e in es]
                    for kk in range(d // L):
                        sl = pl.ds(kk * L, L)
                        for e, ab in zip(es, abs_):
                            x = gX[e, sl] + emX[e, sl]
                            gX[e, sl] = jnp.maximum(x, 0.0) * ab
                    return 0
                lax.fori_loop(0, CB // 2, erow, 0)

            with jax.named_scope("scat"):
                pltpu.sync_copy(gX, agg.at[dstX], add=True)

            if do_meta:
                with jax.named_scope("mi"):
                    issue_meta(X, c + 2)
            if do_next:
                with jax.named_scope("wmgi"):
                    wait_meta(Y, c + 1)
                    issue_gather(Y)

        gA = A[6]

        def zrow(r, _):
            for kk in range(d // L):
                gA[r, pl.ds(kk * L, L)] = jnp.zeros((L,), _F32)
            return 0
        lax.fori_loop(0, CB, zrow, 0)
        for j in range(rpt // CB):
            pltpu.sync_copy(gA, agg.at[pl.ds(sid * rpt + j * CB, CB)])

        plsc.subcore_barrier()

        issue_meta(A, 0)
        wait_meta(A, 0)
        issue_gather(A)
        issue_meta(B, 1)

        def pair(t, _):
            half(A, B, 2 * t, True, True)
            half(B, A, 2 * t + 1, True, True)
            return 0
        lax.fori_loop(0, (nch - 3) // 2, pair, 0)

        half(A, B, nch - 3, True, True)
        half(B, A, nch - 2, False, True)
        half(A, B, nch - 1, False, False)

        plsc.subcore_barrier()

        pltpu.sync_copy(agg.at[pl.ds(sid * rpt, rpt)],
                        out_hbm.at[cid, pl.ds(sid * rpt, rpt)])

    return k(hm, em, src, dst, ae, a_s, a_d)


def kernel(h, edge_index, edge_feat, W_attn, b_attn, W_msg, b_msg, gamma, beta):
    n, d = h.shape
    ne, de = edge_feat.shape
    CB = 80

    wa_sd = W_attn[0, :2 * d].reshape(2, d).T
    wa_e = W_attn[0, 2 * d:].reshape(de, 1)
    Wm_h = W_msg[:, :d].T
    We_m = W_msg[:, d:].T

    hm, asd = _node_precompute(h, Wm_h, wa_sd)
    em, ae = _edge_precompute(edge_feat.T, We_m, wa_e,
                              b_msg.reshape(1, d), b_attn.reshape(1, 1), d)

    src = edge_index[0]
    dst = edge_index[1]
    a_s = asd[:, 0]
    a_d = asd[:, 1]
    ae = ae.reshape(ne)

    parts = _sc_edge_agg(hm, em, src, dst, ae, a_s, a_d)
    return _ln_relu(parts[:, :n], gamma, beta)


# speedup vs baseline: 1.7815x; 1.7815x over previous
import functools
import jax
import jax.numpy as jnp
from jax import lax
from jax.experimental import pallas as pl
from jax.experimental.pallas import tpu as pltpu, tpu_sc as plsc

_F32 = jnp.float32


def _node_precompute(h, Wm_h, wa_sd):
    n, d = h.shape
    nb = 2000

    def body(h_ref, w_ref, wsd_ref, hm_ref, asd_ref):
        hb = h_ref[...]
        hm_ref[...] = jnp.dot(hb, w_ref[...], preferred_element_type=_F32)
        asd_ref[...] = jnp.dot(hb, wsd_ref[...], preferred_element_type=_F32)

    return pl.pallas_call(
        body,
        grid=(n // nb,),
        in_specs=[
            pl.BlockSpec((nb, d), lambda i: (i, 0)),
            pl.BlockSpec((d, d), lambda i: (0, 0)),
            pl.BlockSpec((d, 2), lambda i: (0, 0)),
        ],
        out_specs=[
            pl.BlockSpec((nb, d), lambda i: (i, 0)),
            pl.BlockSpec((nb, 2), lambda i: (i, 0)),
        ],
        out_shape=[
            jax.ShapeDtypeStruct((n, d), _F32),
            jax.ShapeDtypeStruct((n, 2), _F32),
        ],
    )(h, Wm_h, wa_sd)


def _edge_precompute(eft, We_m, wa_e, bm, ba, d):
    de, e = eft.shape
    eb = 16000

    def body(ef_ref, w_ref, we_ref, bm_ref, ba_ref, em_ref, ae_ref):
        efb = ef_ref[...]
        em_ref[...] = lax.dot_general(
            efb, w_ref[...], (((0,), (0,)), ((), ())),
            preferred_element_type=_F32) + bm_ref[...]
        ae_ref[...] = lax.dot_general(
            we_ref[...], efb, (((0,), (0,)), ((), ())),
            preferred_element_type=_F32) + ba_ref[...]

    return pl.pallas_call(
        body,
        grid=(e // eb,),
        in_specs=[
            pl.BlockSpec((de, eb), lambda i: (0, i)),
            pl.BlockSpec((de, d), lambda i: (0, 0)),
            pl.BlockSpec((de, 1), lambda i: (0, 0)),
            pl.BlockSpec((1, d), lambda i: (0, 0)),
            pl.BlockSpec((1, 1), lambda i: (0, 0)),
        ],
        out_specs=[
            pl.BlockSpec((eb, d), lambda i: (i, 0)),
            pl.BlockSpec((1, eb), lambda i: (0, i)),
        ],
        out_shape=[
            jax.ShapeDtypeStruct((e, d), _F32),
            jax.ShapeDtypeStruct((1, e), _F32),
        ],
    )(eft, We_m, wa_e, bm, ba)


def _ln_relu(parts, gamma, beta):
    _, n, d = parts.shape
    nb = 2000

    def body(p_ref, g_ref, b_ref, o_ref):
        x = p_ref[0] + p_ref[1]
        m = jnp.mean(x, axis=-1, keepdims=True)
        c = x - m
        v = jnp.mean(c * c, axis=-1, keepdims=True)
        y = c * lax.rsqrt(v + 1e-5) * g_ref[...] + b_ref[...]
        o_ref[...] = jnp.maximum(y, 0.0)

    return pl.pallas_call(
        body,
        grid=(n // nb,),
        in_specs=[
            pl.BlockSpec((2, nb, d), lambda i: (0, i, 0)),
            pl.BlockSpec((1, d), lambda i: (0, 0)),
            pl.BlockSpec((1, d), lambda i: (0, 0)),
        ],
        out_specs=pl.BlockSpec((nb, d), lambda i: (i, 0)),
        out_shape=jax.ShapeDtypeStruct((n, d), _F32),
    )(parts, gamma.reshape(1, d), beta.reshape(1, d))


def _sc_edge_agg(hm, em, src, dst, ae, a_s, a_d):
    n, d = hm.shape
    ne = em.shape[0]
    NC, NS, L = 2, 16, 16
    NW = NC * NS
    ept = ne // NW
    CB = 80
    nch = ept // CB
    ZR = 32
    npad = ((n + NS * ZR - 1) // (NS * ZR)) * NS * ZR
    rpt = npad // NS

    mesh = plsc.VectorSubcoreMesh(core_axis_name="c", subcore_axis_name="s")

    buf_set = [
        pltpu.VMEM((CB,), jnp.int32),
        pltpu.VMEM((CB,), jnp.int32),
        pltpu.VMEM((CB,), _F32),
        pltpu.VMEM((CB,), _F32),
        pltpu.VMEM((CB,), _F32),
        pltpu.VMEM((CB,), _F32),
        pltpu.VMEM((CB, d), _F32),
        pltpu.VMEM((CB, d), _F32),
        pltpu.VMEM((CB,), jnp.int32),
        pltpu.SemaphoreType.DMA,
        pltpu.SemaphoreType.DMA,
        pltpu.SemaphoreType.DMA,
    ]

    @functools.partial(
        pl.kernel,
        out_type=jax.ShapeDtypeStruct((NC, npad, d), _F32),
        mesh=mesh,
        compiler_params=pltpu.CompilerParams(needs_layout_passes=False),
        scratch_types=buf_set + buf_set + [
            pltpu.VMEM_SHARED((npad, d), _F32),
        ],
    )
    def k(hm_hbm, em_hbm, src_hbm, dst_hbm, ae_hbm, as_hbm, ad_hbm, out_hbm,
          *refs):
        A, B, agg = refs[:12], refs[12:24], refs[24]
        cid = lax.axis_index("c")
        sid = lax.axis_index("s")
        wid = sid * NC + cid
        ebase = wid * ept

        def issue_meta(X, c):
            srcX, dstX, aeX, _, _, _, _, emX, _, msX, _, _ = X
            off = ebase + c * CB
            pltpu.async_copy(src_hbm.at[pl.ds(off, CB)], srcX, msX)
            pltpu.async_copy(dst_hbm.at[pl.ds(off, CB)], dstX, msX)
            pltpu.async_copy(ae_hbm.at[pl.ds(off, CB)], aeX, msX)
            pltpu.async_copy(em_hbm.at[pl.ds(off, CB)], emX, msX)

        def wait_meta(X, c):
            srcX, dstX, aeX, _, _, _, _, emX, _, msX, _, _ = X
            off = ebase + c * CB
            pltpu.make_async_copy(src_hbm.at[pl.ds(off, CB)], srcX, msX).wait()
            pltpu.make_async_copy(dst_hbm.at[pl.ds(off, CB)], dstX, msX).wait()
            pltpu.make_async_copy(ae_hbm.at[pl.ds(off, CB)], aeX, msX).wait()
            pltpu.make_async_copy(em_hbm.at[pl.ds(off, CB)], emX, msX).wait()

        def issue_gather(X):
            srcX, dstX, _, asX, adX, _, gX, _, _, _, gsX, _ = X
            pltpu.async_copy(hm_hbm.at[srcX], gX, gsX)
            pltpu.async_copy(as_hbm.at[srcX], asX, gsX)
            pltpu.async_copy(ad_hbm.at[dstX], adX, gsX)

        def wait_gather(X):
            srcX, dstX, _, asX, adX, _, gX, _, _, _, gsX, _ = X
            pltpu.make_async_copy(hm_hbm.at[srcX], gX, gsX).wait()
            pltpu.make_async_copy(as_hbm.at[srcX], asX, gsX).wait()
            pltpu.make_async_copy(ad_hbm.at[dstX], adX, gsX).wait()

        def wait_scatter(X):
            _, _, _, _, _, _, gX, _, dssX, _, _, ssX = X
            pltpu.make_async_copy(gX, agg.at[dssX], ssX).wait()

        def half(X, Y, c, do_meta, do_next, wait_prev_scatter=True):
            srcX, dstX, aeX, asX, adX, alX, gX, emX, dssX, msX, gsX, ssX = X
            with jax.named_scope("wg"):
                wait_gather(X)

            with jax.named_scope("alpha"):
                def ablk(j, _):
                    sl = pl.ds(j * L, L)
                    logit = asX[sl] + adX[sl] + aeX[sl]
                    alX[sl] = 1.0 / (1.0 + jnp.exp(-logit))
                    return 0
                lax.fori_loop(0, CB // L, ablk, 0)

            with jax.named_scope("erow"):
                @plsc.parallel_loop(0, CB, 1, unroll=4)
                def erow(e):
                    ab = plsc.load_gather(alX,
                                          [jnp.full((L,), e, jnp.int32)])
                    for kk in range(d // L):
                        sl = pl.ds(kk * L, L)
                        x = gX[e, sl] + emX[e, sl]
                        gX[e, sl] = jnp.maximum(x, 0.0) * ab

            with jax.named_scope("scat"):
                pltpu.sync_copy(gX, agg.at[dstX], add=True)

            if do_meta:
                with jax.named_scope("mi"):
                    issue_meta(X, c + 2)
            if do_next:
                with jax.named_scope("wmgi"):
                    wait_meta(Y, c + 1)
                    issue_gather(Y)

        gA = A[6]

        def zrow(r, _):
            for kk in range(d // L):
                gA[r, pl.ds(kk * L, L)] = jnp.zeros((L,), _F32)
            return 0
        lax.fori_loop(0, CB, zrow, 0)
        for j in range(rpt // CB):
            pltpu.sync_copy(gA, agg.at[pl.ds(sid * rpt + j * CB, CB)])

        plsc.subcore_barrier()

        issue_meta(A, 0)
        wait_meta(A, 0)
        issue_gather(A)
        issue_meta(B, 1)

        def pair(t, _):
            half(A, B, 2 * t, True, True)
            half(B, A, 2 * t + 1, True, True)
            return 0
        lax.fori_loop(0, (nch - 3) // 2, pair, 0)

        half(A, B, nch - 3, True, True)
        half(B, A, nch - 2, False, True)
        half(A, B, nch - 1, False, False)

        plsc.subcore_barrier()

        pltpu.sync_copy(agg.at[pl.ds(sid * rpt, rpt)],
                        out_hbm.at[cid, pl.ds(sid * rpt, rpt)])

    return k(hm, em, src, dst, ae, a_s, a_d)


def kernel(h, edge_index, edge_feat, W_attn, b_attn, W_msg, b_msg, gamma, beta):
    n, d = h.shape
    ne, de = edge_feat.shape
    CB = 80

    wa_sd = W_attn[0, :2 * d].reshape(2, d).T
    wa_e = W_attn[0, 2 * d:].reshape(de, 1)
    Wm_h = W_msg[:, :d].T
    We_m = W_msg[:, d:].T

    hm, asd = _node_precompute(h, Wm_h, wa_sd)
    em, ae = _edge_precompute(edge_feat.T, We_m, wa_e,
                              b_msg.reshape(1, d), b_attn.reshape(1, 1), d)

    src = edge_index[0]
    dst = edge_index[1]
    a_s = asd[:, 0]
    a_d = asd[:, 1]
    ae = ae.reshape(ne)

    parts = _sc_edge_agg(hm, em, src, dst, ae, a_s, a_d)
    return _ln_relu(parts[:, :n], gamma, beta)


# speedup vs baseline: 2.0126x; 1.1298x over previous
import functools
import jax
import jax.numpy as jnp
from jax import lax
from jax.experimental import pallas as pl
from jax.experimental.pallas import tpu as pltpu, tpu_sc as plsc

_F32 = jnp.float32


def _node_precompute(h, Wm_h, wa_sd):
    n, d = h.shape
    nb = 2000

    def body(h_ref, w_ref, wsd_ref, hm_ref, asd_ref):
        hb = h_ref[...]
        hm_ref[...] = jnp.dot(hb, w_ref[...], preferred_element_type=_F32)
        asd_ref[...] = jnp.dot(hb, wsd_ref[...], preferred_element_type=_F32)

    return pl.pallas_call(
        body,
        grid=(n // nb,),
        in_specs=[
            pl.BlockSpec((nb, d), lambda i: (i, 0)),
            pl.BlockSpec((d, d), lambda i: (0, 0)),
            pl.BlockSpec((d, 2), lambda i: (0, 0)),
        ],
        out_specs=[
            pl.BlockSpec((nb, d), lambda i: (i, 0)),
            pl.BlockSpec((nb, 2), lambda i: (i, 0)),
        ],
        out_shape=[
            jax.ShapeDtypeStruct((n, d), _F32),
            jax.ShapeDtypeStruct((n, 2), _F32),
        ],
    )(h, Wm_h, wa_sd)


def _edge_precompute(eft, We_m, wa_e, bm, ba, d):
    de, e = eft.shape
    eb = 16000

    def body(ef_ref, w_ref, we_ref, bm_ref, ba_ref, em_ref, ae_ref):
        efb = ef_ref[...]
        em_ref[...] = lax.dot_general(
            efb, w_ref[...], (((0,), (0,)), ((), ())),
            preferred_element_type=_F32) + bm_ref[...]
        ae_ref[...] = lax.dot_general(
            we_ref[...], efb, (((0,), (0,)), ((), ())),
            preferred_element_type=_F32) + ba_ref[...]

    return pl.pallas_call(
        body,
        grid=(e // eb,),
        in_specs=[
            pl.BlockSpec((de, eb), lambda i: (0, i)),
            pl.BlockSpec((de, d), lambda i: (0, 0)),
            pl.BlockSpec((de, 1), lambda i: (0, 0)),
            pl.BlockSpec((1, d), lambda i: (0, 0)),
            pl.BlockSpec((1, 1), lambda i: (0, 0)),
        ],
        out_specs=[
            pl.BlockSpec((eb, d), lambda i: (i, 0)),
            pl.BlockSpec((1, eb), lambda i: (0, i)),
        ],
        out_shape=[
            jax.ShapeDtypeStruct((e, d), _F32),
            jax.ShapeDtypeStruct((1, e), _F32),
        ],
    )(eft, We_m, wa_e, bm, ba)


def _ln_relu(parts, gamma, beta):
    _, n, d = parts.shape
    nb = 2000

    def body(p_ref, g_ref, b_ref, o_ref):
        x = p_ref[0] + p_ref[1]
        m = jnp.mean(x, axis=-1, keepdims=True)
        c = x - m
        v = jnp.mean(c * c, axis=-1, keepdims=True)
        y = c * lax.rsqrt(v + 1e-5) * g_ref[...] + b_ref[...]
        o_ref[...] = jnp.maximum(y, 0.0)

    return pl.pallas_call(
        body,
        grid=(n // nb,),
        in_specs=[
            pl.BlockSpec((2, nb, d), lambda i: (0, i, 0)),
            pl.BlockSpec((1, d), lambda i: (0, 0)),
            pl.BlockSpec((1, d), lambda i: (0, 0)),
        ],
        out_specs=pl.BlockSpec((nb, d), lambda i: (i, 0)),
        out_shape=jax.ShapeDtypeStruct((n, d), _F32),
    )(parts, gamma.reshape(1, d), beta.reshape(1, d))


def _sc_edge_agg(hm, em, src, dst, ae, a_s, a_d):
    n, d = hm.shape
    ne = em.shape[0]
    NC, NS, L = 2, 16, 16
    NW = NC * NS
    ept = ne // NW
    CB = 80
    nch = ept // CB
    ZR = 32
    npad = ((n + NS * ZR - 1) // (NS * ZR)) * NS * ZR
    rpt = npad // NS

    mesh = plsc.VectorSubcoreMesh(core_axis_name="c", subcore_axis_name="s")

    buf_set = [
        pltpu.VMEM((CB,), jnp.int32),
        pltpu.VMEM((CB,), jnp.int32),
        pltpu.VMEM((CB,), _F32),
        pltpu.VMEM((CB,), _F32),
        pltpu.VMEM((CB,), _F32),
        pltpu.VMEM((CB,), _F32),
        pltpu.VMEM((CB, d), _F32),
        pltpu.VMEM((CB, d), _F32),
        pltpu.VMEM((CB,), jnp.int32),
        pltpu.SemaphoreType.DMA,
        pltpu.SemaphoreType.DMA,
        pltpu.SemaphoreType.DMA,
    ]

    @functools.partial(
        pl.kernel,
        out_type=jax.ShapeDtypeStruct((NC, npad, d), _F32),
        mesh=mesh,
        compiler_params=pltpu.CompilerParams(needs_layout_passes=False),
        scratch_types=buf_set + buf_set + [
            pltpu.VMEM_SHARED((npad, d), _F32),
        ],
    )
    def k(hm_hbm, em_hbm, src_hbm, dst_hbm, ae_hbm, as_hbm, ad_hbm, out_hbm,
          *refs):
        A, B, agg = refs[:12], refs[12:24], refs[24]
        cid = lax.axis_index("c")
        sid = lax.axis_index("s")
        wid = sid * NC + cid
        ebase = wid * ept

        def issue_meta(X, c):
            srcX, dstX, aeX, _, _, _, _, emX, _, msX, _, _ = X
            off = ebase + c * CB
            pltpu.async_copy(src_hbm.at[pl.ds(off, CB)], srcX, msX)
            pltpu.async_copy(dst_hbm.at[pl.ds(off, CB)], dstX, msX)
            pltpu.async_copy(ae_hbm.at[pl.ds(off, CB)], aeX, msX)
            pltpu.async_copy(em_hbm.at[pl.ds(off, CB)], emX, msX)

        def wait_meta(X, c):
            srcX, dstX, aeX, _, _, _, _, emX, _, msX, _, _ = X
            off = ebase + c * CB
            pltpu.make_async_copy(src_hbm.at[pl.ds(off, CB)], srcX, msX).wait()
            pltpu.make_async_copy(dst_hbm.at[pl.ds(off, CB)], dstX, msX).wait()
            pltpu.make_async_copy(ae_hbm.at[pl.ds(off, CB)], aeX, msX).wait()
            pltpu.make_async_copy(em_hbm.at[pl.ds(off, CB)], emX, msX).wait()

        def issue_gather(X):
            srcX, dstX, _, asX, adX, _, gX, _, _, _, gsX, _ = X
            pltpu.async_copy(hm_hbm.at[srcX], gX, gsX)
            pltpu.async_copy(as_hbm.at[srcX], asX, gsX)
            pltpu.async_copy(ad_hbm.at[dstX], adX, gsX)

        def wait_gather(X):
            srcX, dstX, _, asX, adX, _, gX, _, _, _, gsX, _ = X
            pltpu.make_async_copy(hm_hbm.at[srcX], gX, gsX).wait()
            pltpu.make_async_copy(as_hbm.at[srcX], asX, gsX).wait()
            pltpu.make_async_copy(ad_hbm.at[dstX], adX, gsX).wait()

        def wait_scatter(X):
            _, _, _, _, _, _, gX, _, dssX, _, _, ssX = X
            pltpu.make_async_copy(gX, agg.at[dssX], ssX).wait()

        def half(X, Y, c, do_meta, do_next, wait_prev_scatter=True):
            srcX, dstX, aeX, asX, adX, alX, gX, emX, dssX, msX, gsX, ssX = X
            wait_gather(X)
            if do_next:
                wait_meta(Y, c + 1)
                issue_gather(Y)

            def ablk(j, _):
                sl = pl.ds(j * L, L)
                logit = asX[sl] + adX[sl] + aeX[sl]
                alX[sl] = 1.0 / (1.0 + jnp.exp(-logit))
                return 0
            lax.fori_loop(0, CB // L, ablk, 0)

            @plsc.parallel_loop(0, CB, 1, unroll=4)
            def erow(e):
                ab = plsc.load_gather(alX, [jnp.full((L,), e, jnp.int32)])
                for kk in range(d // L):
                    sl = pl.ds(kk * L, L)
                    x = gX[e, sl] + emX[e, sl]
                    gX[e, sl] = jnp.maximum(x, 0.0) * ab

            pltpu.sync_copy(gX, agg.at[dstX], add=True)

            if do_meta:
                issue_meta(X, c + 2)

        gA = A[6]

        def zrow(r, _):
            for kk in range(d // L):
                gA[r, pl.ds(kk * L, L)] = jnp.zeros((L,), _F32)
            return 0
        lax.fori_loop(0, CB, zrow, 0)
        for j in range(rpt // CB):
            pltpu.sync_copy(gA, agg.at[pl.ds(sid * rpt + j * CB, CB)])

        plsc.subcore_barrier()

        issue_meta(A, 0)
        wait_meta(A, 0)
        issue_gather(A)
        issue_meta(B, 1)

        def pair(t, _):
            half(A, B, 2 * t, True, True)
            half(B, A, 2 * t + 1, True, True)
            return 0
        lax.fori_loop(0, (nch - 3) // 2, pair, 0)

        half(A, B, nch - 3, True, True)
        half(B, A, nch - 2, False, True)
        half(A, B, nch - 1, False, False)

        plsc.subcore_barrier()

        pltpu.sync_copy(agg.at[pl.ds(sid * rpt, rpt)],
                        out_hbm.at[cid, pl.ds(sid * rpt, rpt)])

    return k(hm, em, src, dst, ae, a_s, a_d)


def kernel(h, edge_index, edge_feat, W_attn, b_attn, W_msg, b_msg, gamma, beta):
    n, d = h.shape
    ne, de = edge_feat.shape
    CB = 80

    wa_sd = W_attn[0, :2 * d].reshape(2, d).T
    wa_e = W_attn[0, 2 * d:].reshape(de, 1)
    Wm_h = W_msg[:, :d].T
    We_m = W_msg[:, d:].T

    hm, asd = _node_precompute(h, Wm_h, wa_sd)
    em, ae = _edge_precompute(edge_feat.T, We_m, wa_e,
                              b_msg.reshape(1, d), b_attn.reshape(1, 1), d)

    src = edge_index[0]
    dst = edge_index[1]
    a_s = asd[:, 0]
    a_d = asd[:, 1]
    ae = ae.reshape(ne)

    parts = _sc_edge_agg(hm, em, src, dst, ae, a_s, a_d)
    return _ln_relu(parts[:, :n], gamma, beta)


# speedup vs baseline: 2.0452x; 1.0162x over previous
import functools
import jax
import jax.numpy as jnp
from jax import lax
from jax.experimental import pallas as pl
from jax.experimental.pallas import tpu as pltpu, tpu_sc as plsc

_F32 = jnp.float32


def _node_precompute(h, Wm_h, wa_sd):
    n, d = h.shape
    nb = 2000

    def body(h_ref, w_ref, wsd_ref, hm_ref, asd_ref):
        hb = h_ref[...]
        hm_ref[...] = jnp.dot(hb, w_ref[...], preferred_element_type=_F32)
        asd_ref[...] = jnp.dot(hb, wsd_ref[...], preferred_element_type=_F32)

    return pl.pallas_call(
        body,
        grid=(n // nb,),
        in_specs=[
            pl.BlockSpec((nb, d), lambda i: (i, 0)),
            pl.BlockSpec((d, d), lambda i: (0, 0)),
            pl.BlockSpec((d, 2), lambda i: (0, 0)),
        ],
        out_specs=[
            pl.BlockSpec((nb, d), lambda i: (i, 0)),
            pl.BlockSpec((nb, 2), lambda i: (i, 0)),
        ],
        out_shape=[
            jax.ShapeDtypeStruct((n, d), _F32),
            jax.ShapeDtypeStruct((n, 2), _F32),
        ],
    )(h, Wm_h, wa_sd)


def _edge_precompute(eft, We_m, wa_e, bm, ba, d):
    de, e = eft.shape
    eb = 16000

    def body(ef_ref, w_ref, we_ref, bm_ref, ba_ref, em_ref, ae_ref):
        efb = ef_ref[...]
        em_ref[...] = lax.dot_general(
            efb, w_ref[...], (((0,), (0,)), ((), ())),
            preferred_element_type=_F32) + bm_ref[...]
        ae_ref[...] = lax.dot_general(
            we_ref[...], efb, (((0,), (0,)), ((), ())),
            preferred_element_type=_F32) + ba_ref[...]

    return pl.pallas_call(
        body,
        grid=(e // eb,),
        in_specs=[
            pl.BlockSpec((de, eb), lambda i: (0, i)),
            pl.BlockSpec((de, d), lambda i: (0, 0)),
            pl.BlockSpec((de, 1), lambda i: (0, 0)),
            pl.BlockSpec((1, d), lambda i: (0, 0)),
            pl.BlockSpec((1, 1), lambda i: (0, 0)),
        ],
        out_specs=[
            pl.BlockSpec((eb, d), lambda i: (i, 0)),
            pl.BlockSpec((1, eb), lambda i: (0, i)),
        ],
        out_shape=[
            jax.ShapeDtypeStruct((e, d), _F32),
            jax.ShapeDtypeStruct((1, e), _F32),
        ],
    )(eft, We_m, wa_e, bm, ba)


def _ln_relu(parts, n, gamma, beta):
    _, _, d = parts.shape
    nb = 2000

    def body(p_ref, g_ref, b_ref, o_ref):
        x = p_ref[0] + p_ref[1]
        m = jnp.mean(x, axis=-1, keepdims=True)
        c = x - m
        v = jnp.mean(c * c, axis=-1, keepdims=True)
        y = c * lax.rsqrt(v + 1e-5) * g_ref[...] + b_ref[...]
        o_ref[...] = jnp.maximum(y, 0.0)

    return pl.pallas_call(
        body,
        grid=(n // nb,),
        in_specs=[
            pl.BlockSpec((2, nb, d), lambda i: (0, i, 0)),
            pl.BlockSpec((1, d), lambda i: (0, 0)),
            pl.BlockSpec((1, d), lambda i: (0, 0)),
        ],
        out_specs=pl.BlockSpec((nb, d), lambda i: (i, 0)),
        out_shape=jax.ShapeDtypeStruct((n, d), _F32),
    )(parts, gamma.reshape(1, d), beta.reshape(1, d))


def _sc_edge_agg(hm, em, src, dst, ae, a_s, a_d):
    n, d = hm.shape
    ne = em.shape[0]
    NC, NS, L = 2, 16, 16
    NW = NC * NS
    ept = ne // NW
    CB = 80
    nch = ept // CB
    ZR = 32
    npad = ((n + NS * ZR - 1) // (NS * ZR)) * NS * ZR
    rpt = npad // NS

    mesh = plsc.VectorSubcoreMesh(core_axis_name="c", subcore_axis_name="s")

    buf_set = [
        pltpu.VMEM((CB,), jnp.int32),
        pltpu.VMEM((CB,), jnp.int32),
        pltpu.VMEM((CB,), _F32),
        pltpu.VMEM((CB,), _F32),
        pltpu.VMEM((CB,), _F32),
        pltpu.VMEM((CB,), _F32),
        pltpu.VMEM((CB, d), _F32),
        pltpu.VMEM((CB, d), _F32),
        pltpu.VMEM((CB,), jnp.int32),
        pltpu.SemaphoreType.DMA,
        pltpu.SemaphoreType.DMA,
        pltpu.SemaphoreType.DMA,
    ]

    @functools.partial(
        pl.kernel,
        out_type=jax.ShapeDtypeStruct((NC, npad, d), _F32),
        mesh=mesh,
        compiler_params=pltpu.CompilerParams(needs_layout_passes=False),
        scratch_types=buf_set + buf_set + [
            pltpu.VMEM_SHARED((npad, d), _F32),
        ],
    )
    def k(hm_hbm, em_hbm, src_hbm, dst_hbm, ae_hbm, as_hbm, ad_hbm, out_hbm,
          *refs):
        A, B, agg = refs[:12], refs[12:24], refs[24]
        cid = lax.axis_index("c")
        sid = lax.axis_index("s")
        wid = sid * NC + cid
        ebase = wid * ept

        def issue_meta(X, c):
            srcX, dstX, aeX, _, _, _, _, emX, _, msX, _, _ = X
            off = ebase + c * CB
            pltpu.async_copy(src_hbm.at[pl.ds(off, CB)], srcX, msX)
            pltpu.async_copy(dst_hbm.at[pl.ds(off, CB)], dstX, msX)
            pltpu.async_copy(ae_hbm.at[pl.ds(off, CB)], aeX, msX)
            pltpu.async_copy(em_hbm.at[pl.ds(off, CB)], emX, msX)

        def wait_meta(X, c):
            srcX, dstX, aeX, _, _, _, _, emX, _, msX, _, _ = X
            off = ebase + c * CB
            pltpu.make_async_copy(src_hbm.at[pl.ds(off, CB)], srcX, msX).wait()
            pltpu.make_async_copy(dst_hbm.at[pl.ds(off, CB)], dstX, msX).wait()
            pltpu.make_async_copy(ae_hbm.at[pl.ds(off, CB)], aeX, msX).wait()
            pltpu.make_async_copy(em_hbm.at[pl.ds(off, CB)], emX, msX).wait()

        def issue_gather(X):
            srcX, dstX, _, asX, adX, _, gX, _, _, _, gsX, _ = X
            pltpu.async_copy(hm_hbm.at[srcX], gX, gsX)
            pltpu.async_copy(as_hbm.at[srcX], asX, gsX)
            pltpu.async_copy(ad_hbm.at[dstX], adX, gsX)

        def wait_gather(X):
            srcX, dstX, _, asX, adX, _, gX, _, _, _, gsX, _ = X
            pltpu.make_async_copy(hm_hbm.at[srcX], gX, gsX).wait()
            pltpu.make_async_copy(as_hbm.at[srcX], asX, gsX).wait()
            pltpu.make_async_copy(ad_hbm.at[dstX], adX, gsX).wait()

        def wait_scatter(X):
            _, _, _, _, _, _, gX, _, dssX, _, _, ssX = X
            pltpu.make_async_copy(gX, agg.at[dssX], ssX).wait()

        def half(X, Y, c, do_meta, do_next, wait_prev_scatter=True):
            srcX, dstX, aeX, asX, adX, alX, gX, emX, dssX, msX, gsX, ssX = X
            wait_gather(X)
            if do_next:
                wait_meta(Y, c + 1)
                issue_gather(Y)

            def ablk(j, _):
                sl = pl.ds(j * L, L)
                logit = asX[sl] + adX[sl] + aeX[sl]
                alX[sl] = 1.0 / (1.0 + jnp.exp(-logit))
                return 0
            lax.fori_loop(0, CB // L, ablk, 0)

            @plsc.parallel_loop(0, CB, 1, unroll=4)
            def erow(e):
                ab = plsc.load_gather(alX, [jnp.full((L,), e, jnp.int32)])
                for kk in range(d // L):
                    sl = pl.ds(kk * L, L)
                    x = gX[e, sl] + emX[e, sl]
                    gX[e, sl] = jnp.maximum(x, 0.0) * ab

            pltpu.sync_copy(gX, agg.at[dstX], add=True)

            if do_meta:
                issue_meta(X, c + 2)

        gA = A[6]

        def zrow(r, _):
            for kk in range(d // L):
                gA[r, pl.ds(kk * L, L)] = jnp.zeros((L,), _F32)
            return 0
        lax.fori_loop(0, CB, zrow, 0)
        for j in range(rpt // CB):
            pltpu.sync_copy(gA, agg.at[pl.ds(sid * rpt + j * CB, CB)])

        plsc.subcore_barrier()

        issue_meta(A, 0)
        wait_meta(A, 0)
        issue_gather(A)
        issue_meta(B, 1)

        def pair(t, _):
            half(A, B, 2 * t, True, True)
            half(B, A, 2 * t + 1, True, True)
            return 0
        lax.fori_loop(0, (nch - 3) // 2, pair, 0)

        half(A, B, nch - 3, True, True)
        half(B, A, nch - 2, False, True)
        half(A, B, nch - 1, False, False)

        plsc.subcore_barrier()

        pltpu.sync_copy(agg.at[pl.ds(sid * rpt, rpt)],
                        out_hbm.at[cid, pl.ds(sid * rpt, rpt)])

    return k(hm, em, src, dst, ae, a_s, a_d)


def kernel(h, edge_index, edge_feat, W_attn, b_attn, W_msg, b_msg, gamma, beta):
    n, d = h.shape
    ne, de = edge_feat.shape
    CB = 80

    wa_sd = W_attn[0, :2 * d].reshape(2, d).T
    wa_e = W_attn[0, 2 * d:].reshape(de, 1)
    Wm_h = W_msg[:, :d].T
    We_m = W_msg[:, d:].T

    hm, asd = _node_precompute(h, Wm_h, wa_sd)
    em, ae = _edge_precompute(edge_feat.T, We_m, wa_e,
                              b_msg.reshape(1, d), b_attn.reshape(1, 1), d)

    src = edge_index[0]
    dst = edge_index[1]
    a_s = asd[:, 0]
    a_d = asd[:, 1]
    ae = ae.reshape(ne)

    parts = _sc_edge_agg(hm, em, src, dst, ae, a_s, a_d)
    return _ln_relu(parts, n, gamma, beta)
